# baseline (speedup 1.0000x reference)
import functools

import jax
import jax.numpy as jnp
from jax import lax
from jax.experimental import pallas as pl
from jax.experimental.pallas import tpu as pltpu
from jax.experimental.pallas import tpu_sc as plsc

_NC = 2
_NS = 16
_NW = _NC * _NS
_MAXLANE = 128
_SG = 16
_RB = 2048


def _sc_degree(dst3d, npad, nblk_w, lane):
    rows_t = npad // _NS
    mesh = plsc.VectorSubcoreMesh(core_axis_name="c", subcore_axis_name="s")

    @functools.partial(
        pl.kernel,
        out_type=jax.ShapeDtypeStruct((_NC, npad), jnp.float32),
        mesh=mesh,
        scratch_types=[
            pltpu.VMEM((_SG, lane), jnp.int32),
            pltpu.VMEM((lane,), jnp.float32),
            pltpu.VMEM((rows_t,), jnp.float32),
            pltpu.VMEM_SHARED((npad,), jnp.float32),
        ],
    )
    def k(dst_hbm, out_hbm, dst_v, ones_v, init_v, acc):
        cid = lax.axis_index("c")
        sid = lax.axis_index("s")
        wid = cid * _NS + sid

        @pl.loop(0, lane, step=16)
        def _(i):
            ones_v[pl.ds(i, 16)] = jnp.ones((16,), jnp.float32)

        val = jnp.where(cid == 0, jnp.float32(1.0), jnp.float32(0.0))

        @pl.loop(0, rows_t, step=16)
        def _(i):
            init_v[pl.ds(i, 16)] = jnp.zeros((16,), jnp.float32) + val

        pltpu.sync_copy(init_v, acc.at[pl.ds(sid * rows_t, rows_t)])
        plsc.subcore_barrier()

        @pl.loop(0, nblk_w // _SG)
        def _(sg):
            pltpu.sync_copy(dst_hbm.at[wid, pl.ds(sg * _SG, _SG)], dst_v)

            @pl.loop(0, _SG)
            def _(j):
                pltpu.sync_copy(ones_v, acc.at[dst_v.at[j]], add=True)

        plsc.subcore_barrier()
        pltpu.sync_copy(acc.at[pl.ds(sid * rows_t, rows_t)],
                        out_hbm.at[cid, pl.ds(sid * rows_t, rows_t)])

    return k(dst3d)


def _sc_segsum(g, src3d, dst3d, nblk_w, lane):
    npad, d = g.shape
    rows_t = npad // _NS
    nchunks = rows_t // _MAXLANE
    nbuf = 2 if d > 64 else 4
    mesh = plsc.VectorSubcoreMesh(core_axis_name="c", subcore_axis_name="s")
    cp = (None if d % 128 == 0
          else pltpu.CompilerParams(use_tc_tiling_on_sc=False))
    dw = max(d, _MAXLANE)

    @functools.partial(
        pl.kernel,
        out_type=jax.ShapeDtypeStruct((_NC, npad, dw), jnp.float32),
        mesh=mesh,
        scratch_types=[
            pltpu.VMEM((_SG, lane), jnp.int32),
            pltpu.VMEM((_SG, lane), jnp.int32),
            pltpu.VMEM((nbuf, _MAXLANE, d), jnp.float32),
            pltpu.VMEM_SHARED((npad, d), jnp.float32),
            pltpu.SemaphoreType.DMA((nbuf,)),
            pltpu.SemaphoreType.DMA((nbuf,)),
        ],
        compiler_params=cp,
    )
    def k(g_hbm, src_hbm, dst_hbm, out_hbm, src_v, dst_v, rows_v, acc, gsem, ssem):
        cid = lax.axis_index("c")
        sid = lax.axis_index("s")
        wid = cid * _NS + sid

        @pl.loop(0, _MAXLANE)
        def _(i):
            @pl.loop(0, d, step=16)
            def _(j):
                rows_v[0, i, pl.ds(j, 16)] = jnp.zeros((16,), jnp.float32)

        base = sid * rows_t

        zcps = [
            pltpu.async_copy(rows_v.at[0],
                             acc.at[pl.ds(base + t * _MAXLANE, _MAXLANE)],
                             gsem.at[0])
            for t in range(nchunks)
        ]
        for cp in zcps:
            cp.wait()

        plsc.subcore_barrier()

        @pl.loop(0, nblk_w // _SG)
        def _(sg):
            icp0 = pltpu.async_copy(src_hbm.at[wid, pl.ds(sg * _SG, _SG)],
                                    src_v, gsem.at[0])
            icp1 = pltpu.async_copy(dst_hbm.at[wid, pl.ds(sg * _SG, _SG)],
                                    dst_v, gsem.at[1 % nbuf])
            icp0.wait()
            icp1.wait()

            @pl.loop(0, _SG // nbuf)
            def _(grp):
                j0 = grp * nbuf
                gathers = []
                for b in range(nbuf):
                    @pl.when(grp > 0)
                    def _():
                        pltpu.make_async_copy(
                            rows_v.at[b, pl.ds(0, lane)],
                            acc.at[dst_v.at[j0 + b]],
                            ssem.at[b]).wait()

                    gathers.append(
                        pltpu.async_copy(g_hbm.at[src_v.at[j0 + b]],
                                         rows_v.at[b, pl.ds(0, lane)],
                                         gsem.at[b]))
                for b in range(nbuf):
                    gathers[b].wait()
                    pltpu.async_copy(rows_v.at[b, pl.ds(0, lane)],
                                     acc.at[dst_v.at[j0 + b]],
                                     ssem.at[b], add=True)

            for b in range(nbuf):
                pltpu.make_async_copy(rows_v.at[b, pl.ds(0, lane)],
                                      acc.at[dst_v.at[b]],
                                      ssem.at[b]).wait()

        plsc.subcore_barrier()

        ocps = [
            pltpu.async_copy(acc.at[pl.ds(base + t * _MAXLANE, _MAXLANE)],
                             out_hbm.at[cid, pl.ds(base + t * _MAXLANE, _MAXLANE),
                                        pl.ds(0, d)],
                             ssem.at[0])
            for t in range(nchunks)
        ]
        for cp in ocps:
            cp.wait()

    return k(g, src3d, dst3d)


def _dinv_of(deg_ref):
    d = deg_ref[0] + deg_ref[1]
    return lax.rsqrt(jnp.maximum(d, 1.0))[:, None]


def _l1_body(x_ref, w_ref, deg_ref, g_ref):
    dinv = _dinv_of(deg_ref)
    y = jnp.dot(x_ref[...], w_ref[...], preferred_element_type=jnp.float32)
    g_ref[...] = y * dinv


def _l2_body(s_ref, g1_ref, deg_ref, b1_ref, w2_ref, g2_ref):
    dinv = _dinv_of(deg_ref)
    agg = dinv * (s_ref[0] + s_ref[1] + g1_ref[...]) + b1_ref[...]
    h = jnp.tanh(agg)
    g2_ref[...] = jnp.dot(h, w2_ref[...], preferred_element_type=jnp.float32) * dinv


def _l3_body(s_ref, g2_ref, deg_ref, b2_ref, o_ref):
    dinv = _dinv_of(deg_ref)
    do = o_ref.shape[-1]
    s = s_ref[0, :, :do] + s_ref[1, :, :do]
    o_ref[...] = dinv * (s + g2_ref[...]) + b2_ref[...]


def _tc_layer1(x, W1, deg2, npad):
    _, di = x.shape
    dh = W1.shape[1]
    return pl.pallas_call(
        _l1_body,
        grid=(npad // _RB,),
        in_specs=[
            pl.BlockSpec((_RB, di), lambda i: (i, 0)),
            pl.BlockSpec((di, dh), lambda i: (0, 0)),
            pl.BlockSpec((_NC, _RB), lambda i: (0, i)),
        ],
        out_specs=pl.BlockSpec((_RB, dh), lambda i: (i, 0)),
        out_shape=jax.ShapeDtypeStruct((npad, dh), jnp.float32),
    )(x, W1, deg2)


def _tc_layer2(s1, g1, deg2, b1, W2):
    npad, dh = g1.shape
    do = W2.shape[1]
    return pl.pallas_call(
        _l2_body,
        grid=(npad // _RB,),
        in_specs=[
            pl.BlockSpec((_NC, _RB, dh), lambda i: (0, i, 0)),
            pl.BlockSpec((_RB, dh), lambda i: (i, 0)),
            pl.BlockSpec((_NC, _RB), lambda i: (0, i)),
            pl.BlockSpec((1, dh), lambda i: (0, 0)),
            pl.BlockSpec((dh, do), lambda i: (0, 0)),
        ],
        out_specs=pl.BlockSpec((_RB, do), lambda i: (i, 0)),
        out_shape=jax.ShapeDtypeStruct((npad, do), jnp.float32),
    )(s1, g1, deg2, b1, W2)


def _tc_layer3(s2, g2, deg2, b2, n):
    npad, do = g2.shape
    dw = s2.shape[-1]
    return pl.pallas_call(
        _l3_body,
        grid=(npad // _RB,),
        in_specs=[
            pl.BlockSpec((_NC, _RB, dw), lambda i: (0, i, 0)),
            pl.BlockSpec((_RB, do), lambda i: (i, 0)),
            pl.BlockSpec((_NC, _RB), lambda i: (0, i)),
            pl.BlockSpec((1, do), lambda i: (0, 0)),
        ],
        out_specs=pl.BlockSpec((_RB, do), lambda i: (i, 0)),
        out_shape=jax.ShapeDtypeStruct((n, do), jnp.float32),
    )(s2, g2, deg2, b2)


def kernel(x, edge_index, W1, b1, W2, b2):
    n, di = x.shape
    dh = W1.shape[1]
    do = W2.shape[1]
    e = edge_index.shape[1]

    blk = _NS * _MAXLANE
    npad = ((n + blk - 1) // blk) * blk

    src = edge_index[0]
    dst = edge_index[1]
    nblk_w = None
    if e % _NW == 0:
        per_w = e // _NW
        cand = (-(-per_w // _MAXLANE) + _SG - 1) // _SG * _SG
        if cand > 0 and per_w % cand == 0 and per_w // cand <= _MAXLANE:
            nblk_w = cand
            lane = per_w // cand
            src3d = src.reshape(_NW, nblk_w, lane)
            dst3d = dst.reshape(_NW, nblk_w, lane)
    if nblk_w is None:
        lane = _MAXLANE
        nblk_w = -(-e // (_NW * lane))
        nblk_w = ((nblk_w + _SG - 1) // _SG) * _SG
        pad = _NW * nblk_w * lane - e
        pad_idx = lax.iota(src.dtype, pad)
        src3d = jnp.concatenate(
            [src, pad_idx % n]).reshape(_NW, nblk_w, lane)
        dst3d = jnp.concatenate(
            [dst, n + pad_idx % (npad - n)]).reshape(_NW, nblk_w, lane)

    deg2 = _sc_degree(dst3d, npad, nblk_w, lane)

    g1 = _tc_layer1(x, W1, deg2, npad)
    s1 = _sc_segsum(g1, src3d, dst3d, nblk_w, lane)
    g2 = _tc_layer2(s1, g1, deg2, b1.reshape(1, dh), W2)
    s2 = _sc_segsum(g2, src3d, dst3d, nblk_w, lane)
    return _tc_layer3(s2, g2, deg2, b2.reshape(1, do), n)


# speedup vs baseline: 1.0162x; 1.0162x over previous
import functools

import jax
import jax.numpy as jnp
from jax import lax
from jax.experimental import pallas as pl
from jax.experimental.pallas import tpu as pltpu
from jax.experimental.pallas import tpu_sc as plsc

_NC = 2
_NS = 16
_NW = _NC * _NS
_MAXLANE = 128
_SG = 16
_RB = 2048


def _sc_degree(dst3d, npad, nblk_w, lane):
    rows_t = npad // _NS
    mesh = plsc.VectorSubcoreMesh(core_axis_name="c", subcore_axis_name="s")

    @functools.partial(
        pl.kernel,
        out_type=jax.ShapeDtypeStruct((_NC, npad), jnp.float32),
        mesh=mesh,
        scratch_types=[
            pltpu.VMEM((_SG, lane), jnp.int32),
            pltpu.VMEM((lane,), jnp.float32),
            pltpu.VMEM((rows_t,), jnp.float32),
            pltpu.VMEM_SHARED((npad,), jnp.float32),
        ],
    )
    def k(dst_hbm, out_hbm, dst_v, ones_v, init_v, acc):
        cid = lax.axis_index("c")
        sid = lax.axis_index("s")
        wid = cid * _NS + sid

        @pl.loop(0, lane, step=16)
        def _(i):
            ones_v[pl.ds(i, 16)] = jnp.ones((16,), jnp.float32)

        val = jnp.where(cid == 0, jnp.float32(1.0), jnp.float32(0.0))

        @pl.loop(0, rows_t, step=16)
        def _(i):
            init_v[pl.ds(i, 16)] = jnp.zeros((16,), jnp.float32) + val

        pltpu.sync_copy(init_v, acc.at[pl.ds(sid * rows_t, rows_t)])
        plsc.subcore_barrier()

        @pl.loop(0, nblk_w // _SG)
        def _(sg):
            pltpu.sync_copy(dst_hbm.at[wid, pl.ds(sg * _SG, _SG)], dst_v)

            @pl.loop(0, _SG)
            def _(j):
                pltpu.sync_copy(ones_v, acc.at[dst_v.at[j]], add=True)

        plsc.subcore_barrier()
        pltpu.sync_copy(acc.at[pl.ds(sid * rows_t, rows_t)],
                        out_hbm.at[cid, pl.ds(sid * rows_t, rows_t)])

    return k(dst3d)


def _sc_segsum(g, src3d, dst3d, nblk_w, lane):
    npad, d = g.shape
    rows_t = npad // _NS
    nchunks = rows_t // _MAXLANE
    nbuf = 2 if d > 64 else 4
    mesh = plsc.VectorSubcoreMesh(core_axis_name="c", subcore_axis_name="s")
    cp = (None if d % 128 == 0
          else pltpu.CompilerParams(use_tc_tiling_on_sc=False))
    dw = max(d, _MAXLANE)

    @functools.partial(
        pl.kernel,
        out_type=jax.ShapeDtypeStruct((_NC, npad, dw), jnp.float32),
        mesh=mesh,
        scratch_types=[
            pltpu.VMEM((_SG, lane), jnp.int32),
            pltpu.VMEM((_SG, lane), jnp.int32),
            pltpu.VMEM((nbuf, _MAXLANE, d), jnp.float32),
            pltpu.VMEM_SHARED((npad, d), jnp.float32),
            pltpu.SemaphoreType.DMA((nbuf,)),
            pltpu.SemaphoreType.DMA((nbuf,)),
        ],
        compiler_params=cp,
    )
    def k(g_hbm, src_hbm, dst_hbm, out_hbm, src_v, dst_v, rows_v, acc, gsem, ssem):
        cid = lax.axis_index("c")
        sid = lax.axis_index("s")
        wid = cid * _NS + sid

        @pl.loop(0, _MAXLANE)
        def _(i):
            @pl.loop(0, d, step=16)
            def _(j):
                rows_v[0, i, pl.ds(j, 16)] = jnp.zeros((16,), jnp.float32)

        base = sid * rows_t

        zcps = [
            pltpu.async_copy(rows_v.at[0],
                             acc.at[pl.ds(base + t * _MAXLANE, _MAXLANE)],
                             gsem.at[0])
            for t in range(nchunks)
        ]
        for cp in zcps:
            cp.wait()

        plsc.subcore_barrier()

        @pl.loop(0, nblk_w // _SG)
        def _(sg):
            icp0 = pltpu.async_copy(src_hbm.at[wid, pl.ds(sg * _SG, _SG)],
                                    src_v, gsem.at[0])
            icp1 = pltpu.async_copy(dst_hbm.at[wid, pl.ds(sg * _SG, _SG)],
                                    dst_v, gsem.at[1 % nbuf])
            icp0.wait()
            icp1.wait()

            @pl.loop(0, _SG // nbuf)
            def _(grp):
                j0 = grp * nbuf
                gathers = []
                for b in range(nbuf):
                    @pl.when(grp > 0)
                    def _():
                        pltpu.make_async_copy(
                            rows_v.at[b, pl.ds(0, lane)],
                            acc.at[dst_v.at[j0 + b]],
                            ssem.at[b]).wait()

                    gathers.append(
                        pltpu.async_copy(g_hbm.at[src_v.at[j0 + b]],
                                         rows_v.at[b, pl.ds(0, lane)],
                                         gsem.at[b]))
                for b in range(nbuf):
                    gathers[b].wait()
                    pltpu.async_copy(rows_v.at[b, pl.ds(0, lane)],
                                     acc.at[dst_v.at[j0 + b]],
                                     ssem.at[b], add=True)

            for b in range(nbuf):
                pltpu.make_async_copy(rows_v.at[b, pl.ds(0, lane)],
                                      acc.at[dst_v.at[b]],
                                      ssem.at[b]).wait()

        plsc.subcore_barrier()

        ocps = [
            pltpu.async_copy(acc.at[pl.ds(base + t * _MAXLANE, _MAXLANE)],
                             out_hbm.at[cid, pl.ds(base + t * _MAXLANE, _MAXLANE),
                                        pl.ds(0, d)],
                             ssem.at[0])
            for t in range(nchunks)
        ]
        for cp in ocps:
            cp.wait()

    return k(g, src3d, dst3d)


def _dinv_of(deg_ref):
    d = deg_ref[0] + deg_ref[1]
    return lax.rsqrt(jnp.maximum(d, 1.0))[:, None]


def _l1_body(x_ref, w_ref, deg_ref, g_ref):
    dinv = _dinv_of(deg_ref)
    y = jnp.dot(x_ref[...], w_ref[...], preferred_element_type=jnp.float32)
    g_ref[...] = y * dinv


def _l2_body(s_ref, g1_ref, deg_ref, b1_ref, w2_ref, g2_ref):
    dinv = _dinv_of(deg_ref)
    agg = dinv * (s_ref[0] + s_ref[1] + g1_ref[...]) + b1_ref[...]
    h = jnp.tanh(agg)
    g2_ref[...] = jnp.dot(h, w2_ref[...], preferred_element_type=jnp.float32) * dinv


def _l3_body(s_ref, g2_ref, deg_ref, b2_ref, o_ref):
    dinv = _dinv_of(deg_ref)
    do = o_ref.shape[-1]
    s = s_ref[0, :, :do] + s_ref[1, :, :do]
    o_ref[...] = dinv * (s + g2_ref[...]) + b2_ref[...]


def _tc_layer1(x, W1, deg2, npad):
    _, di = x.shape
    dh = W1.shape[1]
    return pl.pallas_call(
        _l1_body,
        grid=(npad // _RB,),
        in_specs=[
            pl.BlockSpec((_RB, di), lambda i: (i, 0)),
            pl.BlockSpec((di, dh), lambda i: (0, 0)),
            pl.BlockSpec((_NC, _RB), lambda i: (0, i)),
        ],
        out_specs=pl.BlockSpec((_RB, dh), lambda i: (i, 0)),
        out_shape=jax.ShapeDtypeStruct((npad, dh), jnp.float32),
    )(x, W1, deg2)


def _tc_layer2(s1, g1, deg2, b1, W2):
    npad, dh = g1.shape
    do = W2.shape[1]
    return pl.pallas_call(
        _l2_body,
        grid=(npad // _RB,),
        in_specs=[
            pl.BlockSpec((_NC, _RB, dh), lambda i: (0, i, 0)),
            pl.BlockSpec((_RB, dh), lambda i: (i, 0)),
            pl.BlockSpec((_NC, _RB), lambda i: (0, i)),
            pl.BlockSpec((1, dh), lambda i: (0, 0)),
            pl.BlockSpec((dh, do), lambda i: (0, 0)),
        ],
        out_specs=pl.BlockSpec((_RB, do), lambda i: (i, 0)),
        out_shape=jax.ShapeDtypeStruct((npad, do), jnp.float32),
    )(s1, g1, deg2, b1, W2)


def _tc_layer3(s2, g2, deg2, b2, n):
    npad, do = g2.shape
    dw = s2.shape[-1]
    return pl.pallas_call(
        _l3_body,
        grid=(npad // _RB,),
        in_specs=[
            pl.BlockSpec((_NC, _RB, dw), lambda i: (0, i, 0)),
            pl.BlockSpec((_RB, do), lambda i: (i, 0)),
            pl.BlockSpec((_NC, _RB), lambda i: (0, i)),
            pl.BlockSpec((1, do), lambda i: (0, 0)),
        ],
        out_specs=pl.BlockSpec((_RB, do), lambda i: (i, 0)),
        out_shape=jax.ShapeDtypeStruct((n, do), jnp.float32),
    )(s2, g2, deg2, b2)


def kernel(x, edge_index, W1, b1, W2, b2):
    n, di = x.shape
    dh = W1.shape[1]
    do = W2.shape[1]
    e = edge_index.shape[1]

    blk = _NS * _MAXLANE
    npad = ((n + blk - 1) // blk) * blk

    src = edge_index[0]
    dst = edge_index[1]
    nblk_w = None
    if e % _NW == 0:
        per_w = e // _NW
        cand = (-(-per_w // _MAXLANE) + _SG - 1) // _SG * _SG
        if cand > 0 and per_w % cand == 0 and per_w // cand <= _MAXLANE:
            nblk_w = cand
            lane = per_w // cand
            sd3d = edge_index.reshape(2, _NW, nblk_w, lane)
    if nblk_w is None:
        lane = _MAXLANE
        nblk_w = -(-e // (_NW * lane))
        nblk_w = ((nblk_w + _SG - 1) // _SG) * _SG
        pad = _NW * nblk_w * lane - e
        pad_idx = lax.iota(src.dtype, pad)
        srcp = jnp.concatenate([src, pad_idx % n])
        dstp = jnp.concatenate([dst, n + pad_idx % (npad - n)])
        sd3d = jnp.stack([srcp, dstp]).reshape(2, _NW, nblk_w, lane)
    src3d = sd3d[0]
    dst3d = sd3d[1]

    deg2 = _sc_degree(dst3d, npad, nblk_w, lane)

    g1 = _tc_layer1(x, W1, deg2, npad)
    s1 = _sc_segsum(g1, src3d, dst3d, nblk_w, lane)
    g2 = _tc_layer2(s1, g1, deg2, b1.reshape(1, dh), W2)
    s2 = _sc_segsum(g2, src3d, dst3d, nblk_w, lane)
    return _tc_layer3(s2, g2, deg2, b2.reshape(1, do), n)


# speedup vs baseline: 1.0322x; 1.0158x over previous
import functools

import jax
import jax.numpy as jnp
from jax import lax
from jax.experimental import pallas as pl
from jax.experimental.pallas import tpu as pltpu
from jax.experimental.pallas import tpu_sc as plsc

_NC = 2
_NS = 16
_NW = _NC * _NS
_MAXLANE = 128
_SG = 16
_RB = 2048


def _sc_degree(sd3d, npad, nblk_w, lane):
    rows_t = npad // _NS
    mesh = plsc.VectorSubcoreMesh(core_axis_name="c", subcore_axis_name="s")

    @functools.partial(
        pl.kernel,
        out_type=jax.ShapeDtypeStruct((_NC, npad), jnp.float32),
        mesh=mesh,
        scratch_types=[
            pltpu.VMEM((_SG, lane), jnp.int32),
            pltpu.VMEM((lane,), jnp.float32),
            pltpu.VMEM((rows_t,), jnp.float32),
            pltpu.VMEM_SHARED((npad,), jnp.float32),
        ],
    )
    def k(sd_hbm, out_hbm, dst_v, ones_v, init_v, acc):
        cid = lax.axis_index("c")
        sid = lax.axis_index("s")
        wid = cid * _NS + sid

        @pl.loop(0, lane, step=16)
        def _(i):
            ones_v[pl.ds(i, 16)] = jnp.ones((16,), jnp.float32)

        val = jnp.where(cid == 0, jnp.float32(1.0), jnp.float32(0.0))

        @pl.loop(0, rows_t, step=16)
        def _(i):
            init_v[pl.ds(i, 16)] = jnp.zeros((16,), jnp.float32) + val

        pltpu.sync_copy(init_v, acc.at[pl.ds(sid * rows_t, rows_t)])
        plsc.subcore_barrier()

        @pl.loop(0, nblk_w // _SG)
        def _(sg):
            pltpu.sync_copy(sd_hbm.at[1, wid, pl.ds(sg * _SG, _SG)], dst_v)

            @pl.loop(0, _SG)
            def _(j):
                pltpu.sync_copy(ones_v, acc.at[dst_v.at[j]], add=True)

        plsc.subcore_barrier()
        pltpu.sync_copy(acc.at[pl.ds(sid * rows_t, rows_t)],
                        out_hbm.at[cid, pl.ds(sid * rows_t, rows_t)])

    return k(sd3d)


def _sc_segsum(g, sd3d, nblk_w, lane):
    npad, d = g.shape
    rows_t = npad // _NS
    nchunks = rows_t // _MAXLANE
    nbuf = 2 if d > 64 else 4
    mesh = plsc.VectorSubcoreMesh(core_axis_name="c", subcore_axis_name="s")
    cp = (None if d % 128 == 0
          else pltpu.CompilerParams(use_tc_tiling_on_sc=False))
    dw = max(d, _MAXLANE)

    @functools.partial(
        pl.kernel,
        out_type=jax.ShapeDtypeStruct((_NC, npad, dw), jnp.float32),
        mesh=mesh,
        scratch_types=[
            pltpu.VMEM((_SG, lane), jnp.int32),
            pltpu.VMEM((_SG, lane), jnp.int32),
            pltpu.VMEM((nbuf, _MAXLANE, d), jnp.float32),
            pltpu.VMEM_SHARED((npad, d), jnp.float32),
            pltpu.SemaphoreType.DMA((nbuf,)),
            pltpu.SemaphoreType.DMA((nbuf,)),
        ],
        compiler_params=cp,
    )
    def k(g_hbm, sd_hbm, out_hbm, src_v, dst_v, rows_v, acc, gsem, ssem):
        cid = lax.axis_index("c")
        sid = lax.axis_index("s")
        wid = cid * _NS + sid

        @pl.loop(0, _MAXLANE)
        def _(i):
            @pl.loop(0, d, step=16)
            def _(j):
                rows_v[0, i, pl.ds(j, 16)] = jnp.zeros((16,), jnp.float32)

        base = sid * rows_t

        zcps = [
            pltpu.async_copy(rows_v.at[0],
                             acc.at[pl.ds(base + t * _MAXLANE, _MAXLANE)],
                             gsem.at[0])
            for t in range(nchunks)
        ]
        for cp in zcps:
            cp.wait()

        plsc.subcore_barrier()

        @pl.loop(0, nblk_w // _SG)
        def _(sg):
            icp0 = pltpu.async_copy(sd_hbm.at[0, wid, pl.ds(sg * _SG, _SG)],
                                    src_v, gsem.at[0])
            icp1 = pltpu.async_copy(sd_hbm.at[1, wid, pl.ds(sg * _SG, _SG)],
                                    dst_v, gsem.at[1 % nbuf])
            icp0.wait()
            icp1.wait()

            @pl.loop(0, _SG // nbuf)
            def _(grp):
                j0 = grp * nbuf
                gathers = []
                for b in range(nbuf):
                    @pl.when(grp > 0)
                    def _():
                        pltpu.make_async_copy(
                            rows_v.at[b, pl.ds(0, lane)],
                            acc.at[dst_v.at[j0 + b]],
                            ssem.at[b]).wait()

                    gathers.append(
                        pltpu.async_copy(g_hbm.at[src_v.at[j0 + b]],
                                         rows_v.at[b, pl.ds(0, lane)],
                                         gsem.at[b]))
                for b in range(nbuf):
                    gathers[b].wait()
                    pltpu.async_copy(rows_v.at[b, pl.ds(0, lane)],
                                     acc.at[dst_v.at[j0 + b]],
                                     ssem.at[b], add=True)

            for b in range(nbuf):
                pltpu.make_async_copy(rows_v.at[b, pl.ds(0, lane)],
                                      acc.at[dst_v.at[b]],
                                      ssem.at[b]).wait()

        plsc.subcore_barrier()

        ocps = [
            pltpu.async_copy(acc.at[pl.ds(base + t * _MAXLANE, _MAXLANE)],
                             out_hbm.at[cid, pl.ds(base + t * _MAXLANE, _MAXLANE),
                                        pl.ds(0, d)],
                             ssem.at[0])
            for t in range(nchunks)
        ]
        for cp in ocps:
            cp.wait()

    return k(g, sd3d)


def _dinv_of(deg_ref):
    d = deg_ref[0] + deg_ref[1]
    return lax.rsqrt(jnp.maximum(d, 1.0))[:, None]


def _l1_body(x_ref, w_ref, deg_ref, g_ref):
    dinv = _dinv_of(deg_ref)
    y = jnp.dot(x_ref[...], w_ref[...], preferred_element_type=jnp.float32)
    g_ref[...] = y * dinv


def _l2_body(s_ref, g1_ref, deg_ref, b1_ref, w2_ref, g2_ref):
    dinv = _dinv_of(deg_ref)
    agg = dinv * (s_ref[0] + s_ref[1] + g1_ref[...]) + b1_ref[...]
    h = jnp.tanh(agg)
    g2_ref[...] = jnp.dot(h, w2_ref[...], preferred_element_type=jnp.float32) * dinv


def _l3_body(s_ref, g2_ref, deg_ref, b2_ref, o_ref):
    dinv = _dinv_of(deg_ref)
    do = o_ref.shape[-1]
    s = s_ref[0, :, :do] + s_ref[1, :, :do]
    o_ref[...] = dinv * (s + g2_ref[...]) + b2_ref[...]


def _tc_layer1(x, W1, deg2, npad):
    _, di = x.shape
    dh = W1.shape[1]
    return pl.pallas_call(
        _l1_body,
        grid=(npad // _RB,),
        in_specs=[
            pl.BlockSpec((_RB, di), lambda i: (i, 0)),
            pl.BlockSpec((di, dh), lambda i: (0, 0)),
            pl.BlockSpec((_NC, _RB), lambda i: (0, i)),
        ],
        out_specs=pl.BlockSpec((_RB, dh), lambda i: (i, 0)),
        out_shape=jax.ShapeDtypeStruct((npad, dh), jnp.float32),
    )(x, W1, deg2)


def _tc_layer2(s1, g1, deg2, b1, W2):
    npad, dh = g1.shape
    do = W2.shape[1]
    return pl.pallas_call(
        _l2_body,
        grid=(npad // _RB,),
        in_specs=[
            pl.BlockSpec((_NC, _RB, dh), lambda i: (0, i, 0)),
            pl.BlockSpec((_RB, dh), lambda i: (i, 0)),
            pl.BlockSpec((_NC, _RB), lambda i: (0, i)),
            pl.BlockSpec((1, dh), lambda i: (0, 0)),
            pl.BlockSpec((dh, do), lambda i: (0, 0)),
        ],
        out_specs=pl.BlockSpec((_RB, do), lambda i: (i, 0)),
        out_shape=jax.ShapeDtypeStruct((npad, do), jnp.float32),
    )(s1, g1, deg2, b1, W2)


def _tc_layer3(s2, g2, deg2, b2, n):
    npad, do = g2.shape
    dw = s2.shape[-1]
    return pl.pallas_call(
        _l3_body,
        grid=(npad // _RB,),
        in_specs=[
            pl.BlockSpec((_NC, _RB, dw), lambda i: (0, i, 0)),
            pl.BlockSpec((_RB, do), lambda i: (i, 0)),
            pl.BlockSpec((_NC, _RB), lambda i: (0, i)),
            pl.BlockSpec((1, do), lambda i: (0, 0)),
        ],
        out_specs=pl.BlockSpec((_RB, do), lambda i: (i, 0)),
        out_shape=jax.ShapeDtypeStruct((n, do), jnp.float32),
    )(s2, g2, deg2, b2)


def kernel(x, edge_index, W1, b1, W2, b2):
    n, di = x.shape
    dh = W1.shape[1]
    do = W2.shape[1]
    e = edge_index.shape[1]

    blk = _NS * _MAXLANE
    npad = ((n + blk - 1) // blk) * blk

    src = edge_index[0]
    dst = edge_index[1]
    nblk_w = None
    if e % _NW == 0:
        per_w = e // _NW
        cand = (-(-per_w // _MAXLANE) + _SG - 1) // _SG * _SG
        if cand > 0 and per_w % cand == 0 and per_w // cand <= _MAXLANE:
            nblk_w = cand
            lane = per_w // cand
            sd3d = edge_index.reshape(2, _NW, nblk_w, lane)
    if nblk_w is None:
        lane = _MAXLANE
        nblk_w = -(-e // (_NW * lane))
        nblk_w = ((nblk_w + _SG - 1) // _SG) * _SG
        pad = _NW * nblk_w * lane - e
        pad_idx = lax.iota(src.dtype, pad)
        srcp = jnp.concatenate([src, pad_idx % n])
        dstp = jnp.concatenate([dst, n + pad_idx % (npad - n)])
        sd3d = jnp.stack([srcp, dstp]).reshape(2, _NW, nblk_w, lane)

    deg2 = _sc_degree(sd3d, npad, nblk_w, lane)

    g1 = _tc_layer1(x, W1, deg2, npad)
    s1 = _sc_segsum(g1, sd3d, nblk_w, lane)
    g2 = _tc_layer2(s1, g1, deg2, b1.reshape(1, dh), W2)
    s2 = _sc_segsum(g2, sd3d, nblk_w, lane)
    return _tc_layer3(s2, g2, deg2, b2.reshape(1, do), n)
